# baseline (speedup 1.0000x reference)
import functools

import jax
import jax.numpy as jnp
from jax import lax
from jax.experimental import pallas as pl
from jax.experimental.pallas import tpu as pltpu
from jax.experimental.pallas import tpu_sc as plsc

B = 16384
ED = 128
_C = 2
_BC = B // _C


_NC = 2
_NS = 16
_NW = _NC * _NS
_BPW = _BC // _NW
_IDXW = 128
_NCHUNK = _BPW // _IDXW


def _gather_body(ut, it, uid, iid, xc, idx_v, rows_v, sem):
    wid = lax.axis_index("s") * _NC + lax.axis_index("c")
    base = wid * _BPW
    r0 = wid * _NCHUNK
    pltpu.sync_copy(uid.at[pl.ds(r0, _NCHUNK)], idx_v)
    for j in range(_NCHUNK):
        pltpu.async_copy(ut.at[idx_v.at[j]],
                         rows_v.at[pl.ds(j * _IDXW, _IDXW)], sem).wait()
    pltpu.sync_copy(rows_v, xc.at[pl.ds(base, _BPW), pl.ds(0, ED)])
    pltpu.sync_copy(iid.at[pl.ds(r0, _NCHUNK)], idx_v)
    for j in range(_NCHUNK):
        pltpu.async_copy(it.at[idx_v.at[j]],
                         rows_v.at[pl.ds(j * _IDXW, _IDXW)], sem).wait()
    pltpu.sync_copy(rows_v, xc.at[pl.ds(base, _BPW), pl.ds(ED, ED)])


@functools.cache
def _make_sc_gather():
    return pl.kernel(
        _gather_body,
        out_type=jax.ShapeDtypeStruct((_BC, 2 * ED), jnp.float32),
        mesh=plsc.VectorSubcoreMesh(core_axis_name="c", subcore_axis_name="s"),
        scratch_types=[
            pltpu.VMEM((_NCHUNK, _IDXW), jnp.int32),
            pltpu.VMEM((_BPW, ED), jnp.float32),
            pltpu.SemaphoreType.DMA,
        ],
    )


_TB = 2048


def _mlp_body(xc, e, w01, w0ext, sel16,
              b0, g0, be0, m0, v0,
              w1, b1, g1, be1, m1, v1,
              w2, b2, g2, be2, m2, v2,
              wf, bf, out):
    f32 = jnp.float32
    ew = jnp.dot(sel16[...], w0ext[...], preferred_element_type=f32)

    h = jnp.dot(xc[...], w01[...], preferred_element_type=f32)
    h += jnp.dot(e[...], ew, preferred_element_type=f32)
    s = g0[...] * lax.rsqrt(v0[...] + 1e-5)
    t = (b0[...] - m0[...]) * s + be0[...]
    h = jnp.maximum(h * s + t, 0.0)

    h = jnp.dot(h, w1[...], preferred_element_type=f32)
    s = g1[...] * lax.rsqrt(v1[...] + 1e-5)
    t = (b1[...] - m1[...]) * s + be1[...]
    h = jnp.maximum(h * s + t, 0.0)

    h = jnp.dot(h, w2[...], preferred_element_type=f32)
    s = g2[...] * lax.rsqrt(v2[...] + 1e-5)
    t = (b2[...] - m2[...]) * s + be2[...]
    h = jnp.maximum(h * s + t, 0.0)

    z = jnp.dot(h, wf[...], preferred_element_type=f32)
    z8 = jnp.reshape(z, (_TB // 128, 128)) + bf[...]
    out[...] = 5.0 / (1.0 + jnp.exp(-z8))


def _full(shape):
    return pl.BlockSpec(shape, lambda i: (0, 0))


_mlp = pl.pallas_call(
    _mlp_body,
    grid=(_BC // _TB,),
    in_specs=[
        pl.BlockSpec((_TB, 2 * ED), lambda i: (i, 0)),
        pl.BlockSpec((_TB, 16), lambda i: (i, 0)),
        _full((2 * ED, 1024)),
        _full((8, 1024)),
        _full((16, 8)),
        _full((1, 1024)), _full((1, 1024)), _full((1, 1024)), _full((1, 1024)), _full((1, 1024)),
        _full((1024, 512)),
        _full((1, 512)), _full((1, 512)), _full((1, 512)), _full((1, 512)), _full((1, 512)),
        _full((512, 256)),
        _full((1, 256)), _full((1, 256)), _full((1, 256)), _full((1, 256)), _full((1, 256)),
        _full((2 * ED, 1)),
        _full((1, 1)),
    ],
    out_specs=pl.BlockSpec((_TB // 128, 128), lambda i: (i, 0)),
    out_shape=jax.ShapeDtypeStruct((_BC // 128, 128), jnp.float32),
    compiler_params=pltpu.CompilerParams(
        dimension_semantics=("parallel",),
    ),
)


def kernel(user_ids, item_ids, timestamps, day_of_week,
           user_table, item_table, day_table,
           W0, b0, g0, be0, m0, v0,
           W1, b1, g1, be1, m1, v1,
           W2, b2, g2, be2, m2, v2,
           Wf, bf):
    uid2 = user_ids.astype(jnp.int32).reshape(B // _IDXW, _IDXW)
    iid2 = item_ids.astype(jnp.int32).reshape(B // _IDXW, _IDXW)

    cols = jnp.arange(16)[None, :]
    e = jnp.where(cols == 8, timestamps[:, None],
                  (day_of_week[:, None] == cols).astype(jnp.float32))

    w01 = W0[:2 * ED]
    w0ext = jnp.pad(W0[2 * ED:], ((0, 3), (0, 0)))
    sel16 = (jnp.zeros((16, 8), jnp.float32)
             .at[:7, 1:5].set(day_table).at[8, 0].set(1.0))

    bn = (b0.reshape(1, -1), g0.reshape(1, -1), be0.reshape(1, -1), m0.reshape(1, -1), v0.reshape(1, -1),
          W1,
          b1.reshape(1, -1), g1.reshape(1, -1), be1.reshape(1, -1), m1.reshape(1, -1), v1.reshape(1, -1),
          W2,
          b2.reshape(1, -1), g2.reshape(1, -1), be2.reshape(1, -1), m2.reshape(1, -1), v2.reshape(1, -1),
          Wf, bf.reshape(1, 1))

    gather = _make_sc_gather()
    rows_per_chunk = _BC // _IDXW
    outs = []
    for c in range(_C):
        xc = gather(user_table, item_table,
                    uid2[c * rows_per_chunk:(c + 1) * rows_per_chunk],
                    iid2[c * rows_per_chunk:(c + 1) * rows_per_chunk])
        e_c = e[c * _BC:(c + 1) * _BC]
        outs.append(_mlp(xc, e_c, w01, w0ext, sel16, *bn))
    return jnp.concatenate(outs, axis=0).reshape(B, 1)


# speedup vs baseline: 1.0118x; 1.0118x over previous
import functools

import jax
import jax.numpy as jnp
from jax import lax
from jax.experimental import pallas as pl
from jax.experimental.pallas import tpu as pltpu
from jax.experimental.pallas import tpu_sc as plsc

B = 16384
ED = 128
_C = 2
_BC = B // _C


_NC = 2
_NS = 16
_NW = _NC * _NS
_BPW = _BC // _NW
_IDXW = 128
_NCHUNK = _BPW // _IDXW


def _gather_body(ut, it, uid, iid, xc, idx_u, idx_i, rows_u, rows_i,
                 sem_u, sem_i, sem_s):
    wid = lax.axis_index("s") * _NC + lax.axis_index("c")
    base = wid * _BPW
    r0 = wid * _NCHUNK
    pltpu.sync_copy(uid.at[pl.ds(r0, _NCHUNK)], idx_u)
    pltpu.sync_copy(iid.at[pl.ds(r0, _NCHUNK)], idx_i)
    hu = [pltpu.async_copy(ut.at[idx_u.at[j]],
                           rows_u.at[pl.ds(j * _IDXW, _IDXW)], sem_u)
          for j in range(_NCHUNK)]
    hi = [pltpu.async_copy(it.at[idx_i.at[j]],
                           rows_i.at[pl.ds(j * _IDXW, _IDXW)], sem_i)
          for j in range(_NCHUNK)]
    for h in hu:
        h.wait()
    hs_u = pltpu.async_copy(rows_u, xc.at[pl.ds(base, _BPW), pl.ds(0, ED)], sem_s)
    for h in hi:
        h.wait()
    hs_i = pltpu.async_copy(rows_i, xc.at[pl.ds(base, _BPW), pl.ds(ED, ED)], sem_s)
    hs_u.wait()
    hs_i.wait()


@functools.cache
def _make_sc_gather():
    return pl.kernel(
        _gather_body,
        out_type=jax.ShapeDtypeStruct((_BC, 2 * ED), jnp.float32),
        mesh=plsc.VectorSubcoreMesh(core_axis_name="c", subcore_axis_name="s"),
        scratch_types=[
            pltpu.VMEM((_NCHUNK, _IDXW), jnp.int32),
            pltpu.VMEM((_NCHUNK, _IDXW), jnp.int32),
            pltpu.VMEM((_BPW, ED), jnp.float32),
            pltpu.VMEM((_BPW, ED), jnp.float32),
            pltpu.SemaphoreType.DMA,
            pltpu.SemaphoreType.DMA,
            pltpu.SemaphoreType.DMA,
        ],
    )


_TB = 2048


def _mlp_body(xc, e, w01, w0ext, sel16,
              b0, g0, be0, m0, v0,
              w1, b1, g1, be1, m1, v1,
              w2, b2, g2, be2, m2, v2,
              wf, bf, out):
    f32 = jnp.float32
    ew = jnp.dot(sel16[...], w0ext[...], preferred_element_type=f32)

    h = jnp.dot(xc[...], w01[...], preferred_element_type=f32)
    h += jnp.dot(e[...], ew, preferred_element_type=f32)
    s = g0[...] * lax.rsqrt(v0[...] + 1e-5)
    t = (b0[...] - m0[...]) * s + be0[...]
    h = jnp.maximum(h * s + t, 0.0)

    h = jnp.dot(h, w1[...], preferred_element_type=f32)
    s = g1[...] * lax.rsqrt(v1[...] + 1e-5)
    t = (b1[...] - m1[...]) * s + be1[...]
    h = jnp.maximum(h * s + t, 0.0)

    h = jnp.dot(h, w2[...], preferred_element_type=f32)
    s = g2[...] * lax.rsqrt(v2[...] + 1e-5)
    t = (b2[...] - m2[...]) * s + be2[...]
    h = jnp.maximum(h * s + t, 0.0)

    z = jnp.dot(h, wf[...], preferred_element_type=f32)
    z8 = jnp.reshape(z, (_TB // 128, 128)) + bf[...]
    out[...] = 5.0 / (1.0 + jnp.exp(-z8))


def _full(shape):
    return pl.BlockSpec(shape, lambda i: (0, 0))


_mlp = pl.pallas_call(
    _mlp_body,
    grid=(_BC // _TB,),
    in_specs=[
        pl.BlockSpec((_TB, 2 * ED), lambda i: (i, 0)),
        pl.BlockSpec((_TB, 16), lambda i: (i, 0)),
        _full((2 * ED, 1024)),
        _full((8, 1024)),
        _full((16, 8)),
        _full((1, 1024)), _full((1, 1024)), _full((1, 1024)), _full((1, 1024)), _full((1, 1024)),
        _full((1024, 512)),
        _full((1, 512)), _full((1, 512)), _full((1, 512)), _full((1, 512)), _full((1, 512)),
        _full((512, 256)),
        _full((1, 256)), _full((1, 256)), _full((1, 256)), _full((1, 256)), _full((1, 256)),
        _full((2 * ED, 1)),
        _full((1, 1)),
    ],
    out_specs=pl.BlockSpec((_TB // 128, 128), lambda i: (i, 0)),
    out_shape=jax.ShapeDtypeStruct((_BC // 128, 128), jnp.float32),
    compiler_params=pltpu.CompilerParams(
        dimension_semantics=("parallel",),
    ),
)


def kernel(user_ids, item_ids, timestamps, day_of_week,
           user_table, item_table, day_table,
           W0, b0, g0, be0, m0, v0,
           W1, b1, g1, be1, m1, v1,
           W2, b2, g2, be2, m2, v2,
           Wf, bf):
    uid2 = user_ids.astype(jnp.int32).reshape(B // _IDXW, _IDXW)
    iid2 = item_ids.astype(jnp.int32).reshape(B // _IDXW, _IDXW)

    cols = jnp.arange(16)[None, :]

    w01 = W0[:2 * ED]
    w0ext = jnp.pad(W0[2 * ED:], ((0, 3), (0, 0)))
    sel16 = (jnp.zeros((16, 8), jnp.float32)
             .at[:7, 1:5].set(day_table).at[8, 0].set(1.0))

    bn = (b0.reshape(1, -1), g0.reshape(1, -1), be0.reshape(1, -1), m0.reshape(1, -1), v0.reshape(1, -1),
          W1,
          b1.reshape(1, -1), g1.reshape(1, -1), be1.reshape(1, -1), m1.reshape(1, -1), v1.reshape(1, -1),
          W2,
          b2.reshape(1, -1), g2.reshape(1, -1), be2.reshape(1, -1), m2.reshape(1, -1), v2.reshape(1, -1),
          Wf, bf.reshape(1, 1))

    gather = _make_sc_gather()
    rows_per_chunk = _BC // _IDXW
    outs = []
    for c in range(_C):
        xc = gather(user_table, item_table,
                    uid2[c * rows_per_chunk:(c + 1) * rows_per_chunk],
                    iid2[c * rows_per_chunk:(c + 1) * rows_per_chunk])
        ts_c = timestamps[c * _BC:(c + 1) * _BC]
        dow_c = day_of_week[c * _BC:(c + 1) * _BC]
        e_c = jnp.where(cols == 8, ts_c[:, None],
                        (dow_c[:, None] == cols).astype(jnp.float32))
        outs.append(_mlp(xc, e_c, w01, w0ext, sel16, *bn))
    return jnp.concatenate(outs, axis=0).reshape(B, 1)


# speedup vs baseline: 1.0966x; 1.0838x over previous
import functools

import jax
import jax.numpy as jnp
from jax import lax
from jax.experimental import pallas as pl
from jax.experimental.pallas import tpu as pltpu
from jax.experimental.pallas import tpu_sc as plsc

B = 16384
ED = 128
_C = 2
_BC = B // _C


_NC = 2
_NS = 16
_NW = _NC * _NS
_BPW = _BC // _NW
_IDXW = 128
_NCHUNK = _BPW // _IDXW


def _gather_body(ut, it, uid, iid, ts, dow, xc, et,
                 idx_u, idx_i, rows_u, rows_i, ts_v, dow_v, ebuf,
                 sem_u, sem_i, sem_s):
    wid = lax.axis_index("s") * _NC + lax.axis_index("c")
    base = wid * _BPW
    r0 = wid * _NCHUNK
    pltpu.sync_copy(uid.at[pl.ds(r0, _NCHUNK)], idx_u)
    pltpu.sync_copy(iid.at[pl.ds(r0, _NCHUNK)], idx_i)
    hu = [pltpu.async_copy(ut.at[idx_u.at[j]],
                           rows_u.at[pl.ds(j * _IDXW, _IDXW)], sem_u)
          for j in range(_NCHUNK)]
    hi = [pltpu.async_copy(it.at[idx_i.at[j]],
                           rows_i.at[pl.ds(j * _IDXW, _IDXW)], sem_i)
          for j in range(_NCHUNK)]
    pltpu.sync_copy(ts.at[pl.ds(base, _BPW)], ts_v)
    pltpu.sync_copy(dow.at[pl.ds(base, _BPW)], dow_v)
    zeros16 = jnp.zeros((16,), jnp.float32)
    ones16 = jnp.ones((16,), jnp.float32)
    for g in range(_BPW // 16):
        sl = pl.ds(g * 16, 16)
        dow16 = dow_v[sl]
        ts16 = ts_v[sl]
        for j in range(16):
            if j < 7:
                val = jnp.where(dow16 == j, ones16, zeros16)
            elif j == 8:
                val = ts16
            else:
                val = zeros16
            ebuf[j, sl] = val
    hs_e = pltpu.async_copy(ebuf, et.at[:, pl.ds(base, _BPW)], sem_s)
    for h in hu:
        h.wait()
    hs_u = pltpu.async_copy(rows_u, xc.at[pl.ds(base, _BPW), pl.ds(0, ED)], sem_s)
    for h in hi:
        h.wait()
    hs_i = pltpu.async_copy(rows_i, xc.at[pl.ds(base, _BPW), pl.ds(ED, ED)], sem_s)
    hs_e.wait()
    hs_u.wait()
    hs_i.wait()


@functools.cache
def _make_sc_gather():
    return pl.kernel(
        _gather_body,
        out_type=(jax.ShapeDtypeStruct((_BC, 2 * ED), jnp.float32),
                  jax.ShapeDtypeStruct((16, _BC), jnp.float32)),
        mesh=plsc.VectorSubcoreMesh(core_axis_name="c", subcore_axis_name="s"),
        scratch_types=[
            pltpu.VMEM((_NCHUNK, _IDXW), jnp.int32),
            pltpu.VMEM((_NCHUNK, _IDXW), jnp.int32),
            pltpu.VMEM((_BPW, ED), jnp.float32),
            pltpu.VMEM((_BPW, ED), jnp.float32),
            pltpu.VMEM((_BPW,), jnp.float32),
            pltpu.VMEM((_BPW,), jnp.int32),
            pltpu.VMEM((16, _BPW), jnp.float32),
            pltpu.SemaphoreType.DMA,
            pltpu.SemaphoreType.DMA,
            pltpu.SemaphoreType.DMA,
        ],
    )


_TB = 2048


def _mlp_body(xc, e, w01, w0ext, sel16,
              b0, g0, be0, m0, v0,
              w1, b1, g1, be1, m1, v1,
              w2, b2, g2, be2, m2, v2,
              wf, bf, out):
    f32 = jnp.float32
    ew = jnp.dot(sel16[...], w0ext[...], preferred_element_type=f32)

    h = jnp.dot(xc[...], w01[...], preferred_element_type=f32)
    h += lax.dot_general(e[...], ew, (((0,), (0,)), ((), ())),
                         preferred_element_type=f32)
    s = g0[...] * lax.rsqrt(v0[...] + 1e-5)
    t = (b0[...] - m0[...]) * s + be0[...]
    h = jnp.maximum(h * s + t, 0.0)

    h = jnp.dot(h, w1[...], preferred_element_type=f32)
    s = g1[...] * lax.rsqrt(v1[...] + 1e-5)
    t = (b1[...] - m1[...]) * s + be1[...]
    h = jnp.maximum(h * s + t, 0.0)

    h = jnp.dot(h, w2[...], preferred_element_type=f32)
    s = g2[...] * lax.rsqrt(v2[...] + 1e-5)
    t = (b2[...] - m2[...]) * s + be2[...]
    h = jnp.maximum(h * s + t, 0.0)

    z = jnp.dot(h, wf[...], preferred_element_type=f32)
    z8 = jnp.reshape(z, (_TB // 128, 128)) + bf[...]
    out[...] = 5.0 / (1.0 + jnp.exp(-z8))


def _full(shape):
    return pl.BlockSpec(shape, lambda i: (0, 0))


_mlp = pl.pallas_call(
    _mlp_body,
    grid=(_BC // _TB,),
    in_specs=[
        pl.BlockSpec((_TB, 2 * ED), lambda i: (i, 0)),
        pl.BlockSpec((16, _TB), lambda i: (0, i)),
        _full((2 * ED, 1024)),
        _full((8, 1024)),
        _full((16, 8)),
        _full((1, 1024)), _full((1, 1024)), _full((1, 1024)), _full((1, 1024)), _full((1, 1024)),
        _full((1024, 512)),
        _full((1, 512)), _full((1, 512)), _full((1, 512)), _full((1, 512)), _full((1, 512)),
        _full((512, 256)),
        _full((1, 256)), _full((1, 256)), _full((1, 256)), _full((1, 256)), _full((1, 256)),
        _full((2 * ED, 1)),
        _full((1, 1)),
    ],
    out_specs=pl.BlockSpec((_TB // 128, 128), lambda i: (i, 0)),
    out_shape=jax.ShapeDtypeStruct((_BC // 128, 128), jnp.float32),
    compiler_params=pltpu.CompilerParams(
        dimension_semantics=("parallel",),
    ),
)


def kernel(user_ids, item_ids, timestamps, day_of_week,
           user_table, item_table, day_table,
           W0, b0, g0, be0, m0, v0,
           W1, b1, g1, be1, m1, v1,
           W2, b2, g2, be2, m2, v2,
           Wf, bf):
    uid2 = user_ids.astype(jnp.int32).reshape(B // _IDXW, _IDXW)
    iid2 = item_ids.astype(jnp.int32).reshape(B // _IDXW, _IDXW)
    dow = day_of_week.astype(jnp.int32)

    w01 = W0[:2 * ED]
    w0ext = jnp.pad(W0[2 * ED:], ((0, 3), (0, 0)))
    sel16 = (jnp.zeros((16, 8), jnp.float32)
             .at[:7, 1:5].set(day_table).at[8, 0].set(1.0))

    bn = (b0.reshape(1, -1), g0.reshape(1, -1), be0.reshape(1, -1), m0.reshape(1, -1), v0.reshape(1, -1),
          W1,
          b1.reshape(1, -1), g1.reshape(1, -1), be1.reshape(1, -1), m1.reshape(1, -1), v1.reshape(1, -1),
          W2,
          b2.reshape(1, -1), g2.reshape(1, -1), be2.reshape(1, -1), m2.reshape(1, -1), v2.reshape(1, -1),
          Wf, bf.reshape(1, 1))

    gather = _make_sc_gather()
    rows_per_chunk = _BC // _IDXW
    outs = []
    for c in range(_C):
        xc, e_c = gather(user_table, item_table,
                         uid2[c * rows_per_chunk:(c + 1) * rows_per_chunk],
                         iid2[c * rows_per_chunk:(c + 1) * rows_per_chunk],
                         timestamps[c * _BC:(c + 1) * _BC],
                         dow[c * _BC:(c + 1) * _BC])
        outs.append(_mlp(xc, e_c, w01, w0ext, sel16, *bn))
    return jnp.concatenate(outs, axis=0).reshape(B, 1)


# speedup vs baseline: 1.1215x; 1.0227x over previous
import functools

import jax
import jax.numpy as jnp
from jax import lax
from jax.experimental import pallas as pl
from jax.experimental.pallas import tpu as pltpu
from jax.experimental.pallas import tpu_sc as plsc

B = 16384
ED = 128
_C = 2
_BC = B // _C


_NC = 2
_NS = 16
_NW = _NC * _NS
_BPW = _BC // _NW
_IDXW = 128
_NCHUNK = _BPW // _IDXW


def _gather_body(chunk, ut, it, uid, iid, ts, dow, xc, et,
                 idx_u, idx_i, rows_u, rows_i, ts_v, dow_v, ebuf,
                 sem_u, sem_i, sem_s):
    wid = lax.axis_index("s") * _NC + lax.axis_index("c")
    base = wid * _BPW
    hbase = chunk * _BC + base
    r0 = chunk * (_BC // _IDXW) + wid * _NCHUNK
    pltpu.sync_copy(uid.at[pl.ds(r0, _NCHUNK)], idx_u)
    pltpu.sync_copy(iid.at[pl.ds(r0, _NCHUNK)], idx_i)
    hu = [pltpu.async_copy(ut.at[idx_u.at[j]],
                           rows_u.at[pl.ds(j * _IDXW, _IDXW)], sem_u)
          for j in range(_NCHUNK)]
    hi = [pltpu.async_copy(it.at[idx_i.at[j]],
                           rows_i.at[pl.ds(j * _IDXW, _IDXW)], sem_i)
          for j in range(_NCHUNK)]
    pltpu.sync_copy(ts.at[pl.ds(hbase, _BPW)], ts_v)
    pltpu.sync_copy(dow.at[pl.ds(hbase, _BPW)], dow_v)
    zeros16 = jnp.zeros((16,), jnp.float32)
    ones16 = jnp.ones((16,), jnp.float32)
    for g in range(_BPW // 16):
        sl = pl.ds(g * 16, 16)
        dow16 = dow_v[sl]
        ts16 = ts_v[sl]
        for j in range(16):
            if j < 7:
                val = jnp.where(dow16 == j, ones16, zeros16)
            elif j == 8:
                val = ts16
            else:
                val = zeros16
            ebuf[j, sl] = val
    hs_e = pltpu.async_copy(ebuf, et.at[:, pl.ds(base, _BPW)], sem_s)
    for h in hu:
        h.wait()
    hs_u = pltpu.async_copy(rows_u, xc.at[pl.ds(base, _BPW), pl.ds(0, ED)], sem_s)
    for h in hi:
        h.wait()
    hs_i = pltpu.async_copy(rows_i, xc.at[pl.ds(base, _BPW), pl.ds(ED, ED)], sem_s)
    hs_e.wait()
    hs_u.wait()
    hs_i.wait()


@functools.cache
def _make_sc_gather(chunk):
    return pl.kernel(
        functools.partial(_gather_body, chunk),
        out_type=(jax.ShapeDtypeStruct((_BC, 2 * ED), jnp.float32),
                  jax.ShapeDtypeStruct((16, _BC), jnp.float32)),
        mesh=plsc.VectorSubcoreMesh(core_axis_name="c", subcore_axis_name="s"),
        scratch_types=[
            pltpu.VMEM((_NCHUNK, _IDXW), jnp.int32),
            pltpu.VMEM((_NCHUNK, _IDXW), jnp.int32),
            pltpu.VMEM((_BPW, ED), jnp.float32),
            pltpu.VMEM((_BPW, ED), jnp.float32),
            pltpu.VMEM((_BPW,), jnp.float32),
            pltpu.VMEM((_BPW,), jnp.int32),
            pltpu.VMEM((16, _BPW), jnp.float32),
            pltpu.SemaphoreType.DMA,
            pltpu.SemaphoreType.DMA,
            pltpu.SemaphoreType.DMA,
        ],
    )


_TB = 2048


def _mlp_body(xc, e, w01, w0ext, sel16,
              b0, g0, be0, m0, v0,
              w1, b1, g1, be1, m1, v1,
              w2, b2, g2, be2, m2, v2,
              wf, bf, out):
    f32 = jnp.float32
    ew = jnp.dot(sel16[...], w0ext[...], preferred_element_type=f32)

    h = jnp.dot(xc[...], w01[...], preferred_element_type=f32)
    h += lax.dot_general(e[...], ew, (((0,), (0,)), ((), ())),
                         preferred_element_type=f32)
    s = g0[...] * lax.rsqrt(v0[...] + 1e-5)
    t = (b0[...] - m0[...]) * s + be0[...]
    h = jnp.maximum(h * s + t, 0.0)

    h = jnp.dot(h, w1[...], preferred_element_type=f32)
    s = g1[...] * lax.rsqrt(v1[...] + 1e-5)
    t = (b1[...] - m1[...]) * s + be1[...]
    h = jnp.maximum(h * s + t, 0.0)

    h = jnp.dot(h, w2[...], preferred_element_type=f32)
    s = g2[...] * lax.rsqrt(v2[...] + 1e-5)
    t = (b2[...] - m2[...]) * s + be2[...]
    h = jnp.maximum(h * s + t, 0.0)

    z = jnp.dot(h, wf[...], preferred_element_type=f32)
    z8 = jnp.reshape(z, (_TB // 128, 128)) + bf[...]
    out[...] = 5.0 / (1.0 + jnp.exp(-z8))


def _full(shape):
    return pl.BlockSpec(shape, lambda i: (0, 0))


_mlp = pl.pallas_call(
    _mlp_body,
    grid=(_BC // _TB,),
    in_specs=[
        pl.BlockSpec((_TB, 2 * ED), lambda i: (i, 0)),
        pl.BlockSpec((16, _TB), lambda i: (0, i)),
        _full((2 * ED, 1024)),
        _full((8, 1024)),
        _full((16, 8)),
        _full((1, 1024)), _full((1, 1024)), _full((1, 1024)), _full((1, 1024)), _full((1, 1024)),
        _full((1024, 512)),
        _full((1, 512)), _full((1, 512)), _full((1, 512)), _full((1, 512)), _full((1, 512)),
        _full((512, 256)),
        _full((1, 256)), _full((1, 256)), _full((1, 256)), _full((1, 256)), _full((1, 256)),
        _full((2 * ED, 1)),
        _full((1, 1)),
    ],
    out_specs=pl.BlockSpec((_TB // 128, 128), lambda i: (i, 0)),
    out_shape=jax.ShapeDtypeStruct((_BC // 128, 128), jnp.float32),
    compiler_params=pltpu.CompilerParams(
        dimension_semantics=("parallel",),
    ),
)


def kernel(user_ids, item_ids, timestamps, day_of_week,
           user_table, item_table, day_table,
           W0, b0, g0, be0, m0, v0,
           W1, b1, g1, be1, m1, v1,
           W2, b2, g2, be2, m2, v2,
           Wf, bf):
    uid2 = user_ids.astype(jnp.int32).reshape(B // _IDXW, _IDXW)
    iid2 = item_ids.astype(jnp.int32).reshape(B // _IDXW, _IDXW)
    dow = day_of_week.astype(jnp.int32)

    w01 = W0[:2 * ED]
    w0ext = jnp.pad(W0[2 * ED:], ((0, 3), (0, 0)))
    sel16 = (jnp.zeros((16, 8), jnp.float32)
             .at[:7, 1:5].set(day_table).at[8, 0].set(1.0))

    bn = (b0.reshape(1, -1), g0.reshape(1, -1), be0.reshape(1, -1), m0.reshape(1, -1), v0.reshape(1, -1),
          W1,
          b1.reshape(1, -1), g1.reshape(1, -1), be1.reshape(1, -1), m1.reshape(1, -1), v1.reshape(1, -1),
          W2,
          b2.reshape(1, -1), g2.reshape(1, -1), be2.reshape(1, -1), m2.reshape(1, -1), v2.reshape(1, -1),
          Wf, bf.reshape(1, 1))

    outs = []
    for c in range(_C):
        xc, e_c = _make_sc_gather(c)(user_table, item_table, uid2, iid2,
                                     timestamps, dow)
        outs.append(_mlp(xc, e_c, w01, w0ext, sel16, *bn))
    return jnp.concatenate(outs, axis=0).reshape(B, 1)


# speedup vs baseline: 1.1282x; 1.0060x over previous
import functools

import jax
import jax.numpy as jnp
from jax import lax
from jax.experimental import pallas as pl
from jax.experimental.pallas import tpu as pltpu
from jax.experimental.pallas import tpu_sc as plsc

B = 16384
ED = 128
_C = 2
_BC = B // _C


_NC = 2
_NS = 16
_NW = _NC * _NS
_BPW = _BC // _NW
_IDXW = 128
_NCHUNK = _BPW // _IDXW


def _gather_body(chunk, ut, it, uid, iid, ts, dow, xc, et,
                 idx_u, idx_i, rows_u, rows_i, ts_v, dow_v, ebuf,
                 sem_u, sem_i, sem_s):
    wid = lax.axis_index("s") * _NC + lax.axis_index("c")
    base = wid * _BPW
    hbase = chunk * _BC + base
    r0 = chunk * (_BC // _IDXW) + wid * _NCHUNK
    pltpu.sync_copy(uid.at[pl.ds(r0, _NCHUNK)], idx_u)
    pltpu.sync_copy(iid.at[pl.ds(r0, _NCHUNK)], idx_i)
    hu = [pltpu.async_copy(ut.at[idx_u.at[j]],
                           rows_u.at[pl.ds(j * _IDXW, _IDXW)], sem_u)
          for j in range(_NCHUNK)]
    hi = [pltpu.async_copy(it.at[idx_i.at[j]],
                           rows_i.at[pl.ds(j * _IDXW, _IDXW)], sem_i)
          for j in range(_NCHUNK)]
    pltpu.sync_copy(ts.at[pl.ds(hbase, _BPW)], ts_v)
    pltpu.sync_copy(dow.at[pl.ds(hbase, _BPW)], dow_v)
    zeros16 = jnp.zeros((16,), jnp.float32)
    ones16 = jnp.ones((16,), jnp.float32)
    for g in range(_BPW // 16):
        sl = pl.ds(g * 16, 16)
        dow16 = dow_v[sl]
        ts16 = ts_v[sl]
        for j in range(16):
            if j < 7:
                val = jnp.where(dow16 == j, ones16, zeros16)
            elif j == 8:
                val = ts16
            else:
                val = zeros16
            ebuf[j, sl] = val
    hs_e = pltpu.async_copy(ebuf, et.at[:, pl.ds(base, _BPW)], sem_s)
    for h in hu:
        h.wait()
    hs_u = pltpu.async_copy(rows_u, xc.at[pl.ds(base, _BPW), pl.ds(0, ED)], sem_s)
    for h in hi:
        h.wait()
    hs_i = pltpu.async_copy(rows_i, xc.at[pl.ds(base, _BPW), pl.ds(ED, ED)], sem_s)
    hs_e.wait()
    hs_u.wait()
    hs_i.wait()


@functools.cache
def _make_sc_gather(chunk):
    return pl.kernel(
        functools.partial(_gather_body, chunk),
        out_type=(jax.ShapeDtypeStruct((_BC, 2 * ED), jnp.float32),
                  jax.ShapeDtypeStruct((16, _BC), jnp.float32)),
        mesh=plsc.VectorSubcoreMesh(core_axis_name="c", subcore_axis_name="s"),
        scratch_types=[
            pltpu.VMEM((_NCHUNK, _IDXW), jnp.int32),
            pltpu.VMEM((_NCHUNK, _IDXW), jnp.int32),
            pltpu.VMEM((_BPW, ED), jnp.float32),
            pltpu.VMEM((_BPW, ED), jnp.float32),
            pltpu.VMEM((_BPW,), jnp.float32),
            pltpu.VMEM((_BPW,), jnp.int32),
            pltpu.VMEM((16, _BPW), jnp.float32),
            pltpu.SemaphoreType.DMA,
            pltpu.SemaphoreType.DMA,
            pltpu.SemaphoreType.DMA,
        ],
    )


_TB = 4096


def _mlp_body(xc, e, w01, w0ext, sel16,
              b0, g0, be0, m0, v0,
              w1, b1, g1, be1, m1, v1,
              w2, b2, g2, be2, m2, v2,
              wf, bf, out):
    f32 = jnp.float32
    ew = jnp.dot(sel16[...], w0ext[...], preferred_element_type=f32)

    h = jnp.dot(xc[...], w01[...], preferred_element_type=f32)
    h += lax.dot_general(e[...], ew, (((0,), (0,)), ((), ())),
                         preferred_element_type=f32)
    s = g0[...] * lax.rsqrt(v0[...] + 1e-5)
    t = (b0[...] - m0[...]) * s + be0[...]
    h = jnp.maximum(h * s + t, 0.0)

    h = jnp.dot(h, w1[...], preferred_element_type=f32)
    s = g1[...] * lax.rsqrt(v1[...] + 1e-5)
    t = (b1[...] - m1[...]) * s + be1[...]
    h = jnp.maximum(h * s + t, 0.0)

    h = jnp.dot(h, w2[...], preferred_element_type=f32)
    s = g2[...] * lax.rsqrt(v2[...] + 1e-5)
    t = (b2[...] - m2[...]) * s + be2[...]
    h = jnp.maximum(h * s + t, 0.0)

    z = jnp.dot(h, wf[...], preferred_element_type=f32)
    z8 = jnp.reshape(z, (_TB // 128, 128)) + bf[...]
    out[...] = 5.0 / (1.0 + jnp.exp(-z8))


def _full(shape):
    return pl.BlockSpec(shape, lambda i: (0, 0))


_mlp = pl.pallas_call(
    _mlp_body,
    grid=(_BC // _TB,),
    in_specs=[
        pl.BlockSpec((_TB, 2 * ED), lambda i: (i, 0)),
        pl.BlockSpec((16, _TB), lambda i: (0, i)),
        _full((2 * ED, 1024)),
        _full((8, 1024)),
        _full((16, 8)),
        _full((1, 1024)), _full((1, 1024)), _full((1, 1024)), _full((1, 1024)), _full((1, 1024)),
        _full((1024, 512)),
        _full((1, 512)), _full((1, 512)), _full((1, 512)), _full((1, 512)), _full((1, 512)),
        _full((512, 256)),
        _full((1, 256)), _full((1, 256)), _full((1, 256)), _full((1, 256)), _full((1, 256)),
        _full((2 * ED, 1)),
        _full((1, 1)),
    ],
    out_specs=pl.BlockSpec((_TB // 128, 128), lambda i: (i, 0)),
    out_shape=jax.ShapeDtypeStruct((_BC // 128, 128), jnp.float32),
    compiler_params=pltpu.CompilerParams(
        dimension_semantics=("parallel",),
    ),
)


def kernel(user_ids, item_ids, timestamps, day_of_week,
           user_table, item_table, day_table,
           W0, b0, g0, be0, m0, v0,
           W1, b1, g1, be1, m1, v1,
           W2, b2, g2, be2, m2, v2,
           Wf, bf):
    uid2 = user_ids.astype(jnp.int32).reshape(B // _IDXW, _IDXW)
    iid2 = item_ids.astype(jnp.int32).reshape(B // _IDXW, _IDXW)
    dow = day_of_week.astype(jnp.int32)

    w01 = W0[:2 * ED]
    w0ext = jnp.pad(W0[2 * ED:], ((0, 3), (0, 0)))
    sel16 = (jnp.zeros((16, 8), jnp.float32)
             .at[:7, 1:5].set(day_table).at[8, 0].set(1.0))

    bn = (b0.reshape(1, -1), g0.reshape(1, -1), be0.reshape(1, -1), m0.reshape(1, -1), v0.reshape(1, -1),
          W1,
          b1.reshape(1, -1), g1.reshape(1, -1), be1.reshape(1, -1), m1.reshape(1, -1), v1.reshape(1, -1),
          W2,
          b2.reshape(1, -1), g2.reshape(1, -1), be2.reshape(1, -1), m2.reshape(1, -1), v2.reshape(1, -1),
          Wf, bf.reshape(1, 1))

    outs = []
    for c in range(_C):
        xc, e_c = _make_sc_gather(c)(user_table, item_table, uid2, iid2,
                                     timestamps, dow)
        outs.append(_mlp(xc, e_c, w01, w0ext, sel16, *bn))
    return jnp.concatenate(outs, axis=0).reshape(B, 1)
